# baseline (speedup 1.0000x reference)
import functools

import jax
import jax.numpy as jnp
from jax import lax
from jax.experimental import pallas as pl
from jax.experimental.pallas import tpu as pltpu
from jax.experimental.pallas import tpu_sc as plsc

NC = 2
NS = 16
NW = NC * NS
CHUNK = 80
DEGW = 16


def _sc_mesh():
    return plsc.VectorSubcoreMesh(
        core_axis_name="c", subcore_axis_name="s", num_cores=NC, num_subcores=NS)


_NBUF = 6
_LEAD = _NBUF - 2


def _sc_degree(dst, n_nodes):
    e = dst.shape[0]
    epw = e // NW
    nch = epw // CHUNK
    ones = jnp.ones((CHUNK, DEGW), jnp.float32)
    zeros = jnp.zeros((n_nodes, DEGW), jnp.float32)
    dst3 = dst.reshape(NW, nch, CHUNK)

    @functools.partial(
        pl.kernel,
        mesh=_sc_mesh(),
        out_type=jax.ShapeDtypeStruct((NC, n_nodes, DEGW), jnp.float32),
        scratch_types=[
            pltpu.VMEM((nch, CHUNK), jnp.int32),
            pltpu.VMEM((CHUNK, DEGW), jnp.float32),
            pltpu.VMEM_SHARED((n_nodes, DEGW), jnp.float32),
            pltpu.SemaphoreType.DMA,
        ],
        compiler_params=pltpu.CompilerParams(use_tc_tiling_on_sc=False),
    )
    def deg_kernel(dst_hbm, ones_hbm, zeros_hbm, out_hbm, idx_v, ones_v, acc_sh,
                   sem):
        c = lax.axis_index("c")
        s = lax.axis_index("s")
        wid = c * NS + s

        @pl.when(s == 0)
        def _zero():
            pltpu.sync_copy(zeros_hbm, acc_sh)

        pltpu.sync_copy(dst_hbm.at[wid], idx_v)
        pltpu.sync_copy(ones_hbm, ones_v)
        plsc.subcore_barrier()

        lag = 2 * _NBUF

        def wait_one():
            pltpu.make_async_copy(ones_v, acc_sh.at[idx_v.at[0]], sem).wait()

        def body(k, carry):
            pltpu.async_copy(ones_v, acc_sh.at[idx_v.at[k]], sem, add=True)

            @pl.when(k >= lag)
            def _():
                wait_one()

            return carry

        lax.fori_loop(0, nch, body, 0)
        for _ in range(lag):
            wait_one()
        plsc.subcore_barrier()

        @pl.when(s == 0)
        def _out():
            pltpu.sync_copy(acc_sh, out_hbm.at[c])

    return deg_kernel(dst3, ones, zeros)


def _sc_aggregate(table, src4, dst4, n_nodes):
    feat = table.shape[1]
    nch = src4[0].shape[1]
    zeros = jnp.zeros((n_nodes, feat), jnp.float32)

    @functools.partial(
        pl.kernel,
        mesh=_sc_mesh(),
        out_type=jax.ShapeDtypeStruct((NC, n_nodes, feat), jnp.float32),
        scratch_types=[
            pltpu.VMEM((nch, CHUNK), jnp.int32),
            pltpu.VMEM((nch, CHUNK), jnp.int32),
            pltpu.VMEM((_NBUF, CHUNK, feat), jnp.float32),
            pltpu.VMEM_SHARED((n_nodes, feat), jnp.float32),
            [pltpu.SemaphoreType.DMA] * _NBUF,
            [pltpu.SemaphoreType.DMA] * _NBUF,
        ],
        compiler_params=pltpu.CompilerParams(use_tc_tiling_on_sc=False),
    )
    def agg_kernel(g_hbm, srca_hbm, srcb_hbm, dsta_hbm, dstb_hbm, zeros_hbm,
                   out_hbm, src_v, dst_v, rows_v, acc_sh, sem_g, sem_s):
        c = lax.axis_index("c")
        s = lax.axis_index("s")

        @pl.when(s == 0)
        def _zero():
            pltpu.sync_copy(zeros_hbm, acc_sh)

        @pl.when(c == 0)
        def _load_a():
            pltpu.sync_copy(srca_hbm.at[s], src_v)
            pltpu.sync_copy(dsta_hbm.at[s], dst_v)

        @pl.when(c == 1)
        def _load_b():
            pltpu.sync_copy(srcb_hbm.at[s], src_v)
            pltpu.sync_copy(dstb_hbm.at[s], dst_v)

        plsc.subcore_barrier()

        def start_gather(k, b):
            pltpu.async_copy(g_hbm.at[src_v.at[k]], rows_v.at[b], sem_g[b])

        def wait_gather(k, b):
            pltpu.make_async_copy(
                g_hbm.at[src_v.at[k]], rows_v.at[b], sem_g[b]).wait()

        def start_scatter(k, b):
            pltpu.async_copy(rows_v.at[b], acc_sh.at[dst_v.at[k]], sem_s[b],
                             add=True)

        def wait_scatter(k, b):
            pltpu.make_async_copy(
                rows_v.at[b], acc_sh.at[dst_v.at[k]], sem_s[b]).wait()

        for b in range(_LEAD):
            start_gather(b, b)

        def body(k, carry):
            for b in range(_NBUF):
                @pl.when(k % _NBUF == b)
                def _():
                    bp = (b + _LEAD) % _NBUF

                    @pl.when(k + _LEAD < nch)
                    def _():
                        @pl.when(k >= _NBUF - _LEAD)
                        def _():
                            wait_scatter(k - (_NBUF - _LEAD), bp)
                        start_gather(k + _LEAD, bp)

                    wait_gather(k, b)
                    start_scatter(k, b)
            return carry

        lax.fori_loop(0, nch, body, 0)
        for b in range(_NBUF):
            wait_scatter(0, b)
        plsc.subcore_barrier()

        @pl.when(s == 0)
        def _out():
            pltpu.sync_copy(acc_sh, out_hbm.at[c])

    return agg_kernel(table, src4[0], src4[1], dst4[0], dst4[1], zeros)


def _agg_edge(g, src, dst):
    n_nodes = g.shape[0]
    e = src.shape[0]
    h = e // 2
    nch = e // NW // CHUNK
    src4 = (src[:h].reshape(NS, nch, CHUNK), src[h:].reshape(NS, nch, CHUNK))
    dst4 = (dst[:h].reshape(NS, nch, CHUNK), dst[h:].reshape(NS, nch, CHUNK))
    return _sc_aggregate(g, src4, dst4, n_nodes)


def _agg_feat(gs, src, dst):
    _, n_nodes, f2 = gs.shape
    e = src.shape[0]
    nch = e // NS // CHUNK
    table = gs.reshape(NC * n_nodes, f2)
    srca = src.reshape(NS, nch, CHUNK)
    srcb = (src + n_nodes).reshape(NS, nch, CHUNK)
    dstr = dst.reshape(NS, nch, CHUNK)
    return _sc_aggregate(table, (srca, srcb), (dstr, dstr), n_nodes)


_BN = 10000


def _dinv_from_partials(degp_block):
    deg = degp_block[0, :, 0] + degp_block[1, :, 0] + 1.0
    return lax.rsqrt(deg)


def _tc_matmul(x, w):
    n, din = x.shape
    dh = w.shape[1]

    def body(x_ref, w_ref, h_ref):
        h_ref[...] = jnp.dot(x_ref[...], w_ref[...],
                             preferred_element_type=jnp.float32)

    return pl.pallas_call(
        body,
        grid=(n // _BN,),
        in_specs=[
            pl.BlockSpec((_BN, din), lambda i: (i, 0)),
            pl.BlockSpec((din, dh), lambda i: (0, 0)),
        ],
        out_specs=pl.BlockSpec((_BN, dh), lambda i: (i, 0)),
        out_shape=jax.ShapeDtypeStruct((n, dh), jnp.float32),
    )(x, w)


def _tc_scale(h, degp):
    n, dh = h.shape
    f2 = dh // 2

    def body(h_ref, degp_ref, g_ref):
        dinv = _dinv_from_partials(degp_ref[...])
        g = h_ref[...] * dinv[:, None]
        g_ref[0] = g[:, :f2]
        g_ref[1] = g[:, f2:]

    return pl.pallas_call(
        body,
        grid=(n // _BN,),
        in_specs=[
            pl.BlockSpec((_BN, dh), lambda i: (i, 0)),
            pl.BlockSpec((NC, _BN, DEGW), lambda i: (0, i, 0)),
        ],
        out_specs=pl.BlockSpec((NC, _BN, f2), lambda i: (0, i, 0)),
        out_shape=jax.ShapeDtypeStruct((NC, n, f2), jnp.float32),
    )(h, degp)


def _tc_mid(acc1p, g1s, degp, b1, w2):
    _, n, f2 = g1s.shape
    dh = 2 * f2
    dout = w2.shape[1]

    def body(acc_ref, g1_ref, degp_ref, b1_ref, w2_ref, g2_ref):
        dinv = _dinv_from_partials(degp_ref[...])
        acc = acc_ref[...]
        g1 = g1_ref[...]
        s = jnp.concatenate([acc[0] + g1[0], acc[1] + g1[1]], axis=1)
        h = jnp.maximum(s * dinv[:, None] + b1_ref[...][None, :], 0.0)
        g2 = jnp.dot(h, w2_ref[...], preferred_element_type=jnp.float32)
        g2_ref[...] = g2 * dinv[:, None]

    return pl.pallas_call(
        body,
        grid=(n // _BN,),
        in_specs=[
            pl.BlockSpec((NC, _BN, f2), lambda i: (0, i, 0)),
            pl.BlockSpec((NC, _BN, f2), lambda i: (0, i, 0)),
            pl.BlockSpec((NC, _BN, DEGW), lambda i: (0, i, 0)),
            pl.BlockSpec((dh,), lambda i: (0,)),
            pl.BlockSpec((dh, dout), lambda i: (0, 0)),
        ],
        out_specs=pl.BlockSpec((_BN, dout), lambda i: (i, 0)),
        out_shape=jax.ShapeDtypeStruct((n, dout), jnp.float32),
    )(acc1p, g1s, degp, b1, w2)


def _tc_final(acc2p, g2, degp, b2):
    n, dout = g2.shape

    def body(acc_ref, g2_ref, degp_ref, b2_ref, logits_ref, preds_ref, z_ref):
        dinv = _dinv_from_partials(degp_ref[...])
        acc = acc_ref[...]
        z = (acc[0] + acc[1] + g2_ref[...]) * dinv[:, None] + b2_ref[...][None, :]
        m = jnp.max(z, axis=1, keepdims=True)
        ez = jnp.exp(z - m)
        logits = ez / jnp.sum(ez, axis=1, keepdims=True)
        ids = lax.broadcasted_iota(jnp.int32, z.shape, 1)
        cand = jnp.where(z == m, ids, dout)
        z_ref[...] = z
        logits_ref[...] = logits
        preds_ref[...] = jnp.min(cand, axis=1, keepdims=True)

    return pl.pallas_call(
        body,
        grid=(n // _BN,),
        in_specs=[
            pl.BlockSpec((NC, _BN, dout), lambda i: (0, i, 0)),
            pl.BlockSpec((_BN, dout), lambda i: (i, 0)),
            pl.BlockSpec((NC, _BN, DEGW), lambda i: (0, i, 0)),
            pl.BlockSpec((dout,), lambda i: (0,)),
        ],
        out_specs=[
            pl.BlockSpec((_BN, dout), lambda i: (i, 0)),
            pl.BlockSpec((_BN, 1), lambda i: (i, 0)),
            pl.BlockSpec((_BN, dout), lambda i: (i, 0)),
        ],
        out_shape=[
            jax.ShapeDtypeStruct((n, dout), jnp.float32),
            jax.ShapeDtypeStruct((n, 1), jnp.int32),
            jax.ShapeDtypeStruct((n, dout), jnp.float32),
        ],
    )(acc2p, g2, degp, b2)


def kernel(x, edge_index, W1, b1, W2, b2):
    n = x.shape[0]
    src = edge_index[0].astype(jnp.int32)
    dst = edge_index[1].astype(jnp.int32)

    degp = _sc_degree(dst, n)
    h1 = _tc_matmul(x, W1)
    g1s = _tc_scale(h1, degp)
    acc1 = _agg_feat(g1s, src, dst)
    g2 = _tc_mid(acc1, g1s, degp, b1, W2)
    acc2 = _agg_edge(g2, src, dst)
    logits, preds, z = _tc_final(acc2, g2, degp, b2)
    return (logits, preds.reshape(n), z)


# speedup vs baseline: 1.0292x; 1.0292x over previous
import functools

import jax
import jax.numpy as jnp
from jax import lax
from jax.experimental import pallas as pl
from jax.experimental.pallas import tpu as pltpu
from jax.experimental.pallas import tpu_sc as plsc

NC = 2
NS = 16
NW = NC * NS
CHUNK = 80
DEGW = 16


def _sc_mesh():
    return plsc.VectorSubcoreMesh(
        core_axis_name="c", subcore_axis_name="s", num_cores=NC, num_subcores=NS)


_NBUF = 6
_LEAD = _NBUF - 2


def _sc_degree(ei4, n_nodes):
    nch = ei4.shape[3]
    ones = jnp.ones((CHUNK, DEGW), jnp.float32)
    zeros = jnp.zeros((n_nodes, DEGW), jnp.float32)

    @functools.partial(
        pl.kernel,
        mesh=_sc_mesh(),
        out_type=jax.ShapeDtypeStruct((NC, n_nodes, DEGW), jnp.float32),
        scratch_types=[
            pltpu.VMEM((nch, CHUNK), jnp.int32),
            pltpu.VMEM((CHUNK, DEGW), jnp.float32),
            pltpu.VMEM_SHARED((n_nodes, DEGW), jnp.float32),
            pltpu.SemaphoreType.DMA,
        ],
        compiler_params=pltpu.CompilerParams(use_tc_tiling_on_sc=False),
    )
    def deg_kernel(ei_hbm, ones_hbm, zeros_hbm, out_hbm, idx_v, ones_v, acc_sh,
                   sem):
        c = lax.axis_index("c")
        s = lax.axis_index("s")

        @pl.when(s == 0)
        def _zero():
            pltpu.sync_copy(zeros_hbm, acc_sh)

        pltpu.sync_copy(ei_hbm.at[1, c, s], idx_v)
        pltpu.sync_copy(ones_hbm, ones_v)
        plsc.subcore_barrier()

        lag = 2 * _NBUF

        def wait_one():
            pltpu.make_async_copy(ones_v, acc_sh.at[idx_v.at[0]], sem).wait()

        def body(k, carry):
            pltpu.async_copy(ones_v, acc_sh.at[idx_v.at[k]], sem, add=True)

            @pl.when(k >= lag)
            def _():
                wait_one()

            return carry

        lax.fori_loop(0, nch, body, 0)
        for _ in range(lag):
            wait_one()
        plsc.subcore_barrier()

        @pl.when(s == 0)
        def _out():
            pltpu.sync_copy(acc_sh, out_hbm.at[c])

    return deg_kernel(ei4, ones, zeros)


def _sc_aggregate(table, ei4, srcp4, n_nodes, feat_mode):
    feat = table.shape[1]
    nh = ei4.shape[3]
    nch = 2 * nh if feat_mode else nh
    zeros = jnp.zeros((n_nodes, feat), jnp.float32)

    @functools.partial(
        pl.kernel,
        mesh=_sc_mesh(),
        out_type=jax.ShapeDtypeStruct((NC, n_nodes, feat), jnp.float32),
        scratch_types=[
            pltpu.VMEM((nch, CHUNK), jnp.int32),
            pltpu.VMEM((nch, CHUNK), jnp.int32),
            pltpu.VMEM((_NBUF, CHUNK, feat), jnp.float32),
            pltpu.VMEM_SHARED((n_nodes, feat), jnp.float32),
            [pltpu.SemaphoreType.DMA] * _NBUF,
            [pltpu.SemaphoreType.DMA] * _NBUF,
        ],
        compiler_params=pltpu.CompilerParams(use_tc_tiling_on_sc=False),
    )
    def agg_kernel(g_hbm, ei_hbm, srcp_hbm, zeros_hbm,
                   out_hbm, src_v, dst_v, rows_v, acc_sh, sem_g, sem_s):
        c = lax.axis_index("c")
        s = lax.axis_index("s")

        @pl.when(s == 0)
        def _zero():
            pltpu.sync_copy(zeros_hbm, acc_sh)

        if feat_mode:
            @pl.when(c == 0)
            def _load_src0():
                pltpu.sync_copy(ei_hbm.at[0, 0, s], src_v.at[pl.ds(0, nh)])
                pltpu.sync_copy(ei_hbm.at[0, 1, s], src_v.at[pl.ds(nh, nh)])

            @pl.when(c == 1)
            def _load_src1():
                pltpu.sync_copy(srcp_hbm.at[0, s], src_v.at[pl.ds(0, nh)])
                pltpu.sync_copy(srcp_hbm.at[1, s], src_v.at[pl.ds(nh, nh)])

            pltpu.sync_copy(ei_hbm.at[1, 0, s], dst_v.at[pl.ds(0, nh)])
            pltpu.sync_copy(ei_hbm.at[1, 1, s], dst_v.at[pl.ds(nh, nh)])
        else:
            pltpu.sync_copy(ei_hbm.at[0, c, s], src_v)
            pltpu.sync_copy(ei_hbm.at[1, c, s], dst_v)

        plsc.subcore_barrier()

        def start_gather(k, b):
            pltpu.async_copy(g_hbm.at[src_v.at[k]], rows_v.at[b], sem_g[b])

        def wait_gather(k, b):
            pltpu.make_async_copy(
                g_hbm.at[src_v.at[k]], rows_v.at[b], sem_g[b]).wait()

        def start_scatter(k, b):
            pltpu.async_copy(rows_v.at[b], acc_sh.at[dst_v.at[k]], sem_s[b],
                             add=True)

        def wait_scatter(k, b):
            pltpu.make_async_copy(
                rows_v.at[b], acc_sh.at[dst_v.at[k]], sem_s[b]).wait()

        for b in range(_LEAD):
            start_gather(b, b)

        def body(k, carry):
            for b in range(_NBUF):
                @pl.when(k % _NBUF == b)
                def _():
                    bp = (b + _LEAD) % _NBUF

                    @pl.when(k + _LEAD < nch)
                    def _():
                        @pl.when(k >= _NBUF - _LEAD)
                        def _():
                            wait_scatter(k - (_NBUF - _LEAD), bp)
                        start_gather(k + _LEAD, bp)

                    wait_gather(k, b)
                    start_scatter(k, b)
            return carry

        lax.fori_loop(0, nch, body, 0)
        for b in range(_NBUF):
            wait_scatter(0, b)
        plsc.subcore_barrier()

        @pl.when(s == 0)
        def _out():
            pltpu.sync_copy(acc_sh, out_hbm.at[c])

    return agg_kernel(table, ei4, srcp4, zeros)


_BN = 2000


def _dinv_from_partials(degp_block):
    deg = degp_block[0, :, 0] + degp_block[1, :, 0] + 1.0
    return lax.rsqrt(deg)


def _tc_matmul(x, w):
    n, din = x.shape
    dh = w.shape[1]

    def body(x_ref, w_ref, h_ref):
        h_ref[...] = jnp.dot(x_ref[...], w_ref[...],
                             preferred_element_type=jnp.float32)

    return pl.pallas_call(
        body,
        grid=(n // _BN,),
        in_specs=[
            pl.BlockSpec((_BN, din), lambda i: (i, 0)),
            pl.BlockSpec((din, dh), lambda i: (0, 0)),
        ],
        out_specs=pl.BlockSpec((_BN, dh), lambda i: (i, 0)),
        out_shape=jax.ShapeDtypeStruct((n, dh), jnp.float32),
    )(x, w)


def _tc_scale(h, degp):
    n, dh = h.shape
    f2 = dh // 2

    def body(h_ref, degp_ref, g_ref):
        dinv = _dinv_from_partials(degp_ref[...])
        g = h_ref[...] * dinv[:, None]
        g_ref[0] = g[:, :f2]
        g_ref[1] = g[:, f2:]

    return pl.pallas_call(
        body,
        grid=(n // _BN,),
        in_specs=[
            pl.BlockSpec((_BN, dh), lambda i: (i, 0)),
            pl.BlockSpec((NC, _BN, DEGW), lambda i: (0, i, 0)),
        ],
        out_specs=pl.BlockSpec((NC, _BN, f2), lambda i: (0, i, 0)),
        out_shape=jax.ShapeDtypeStruct((NC, n, f2), jnp.float32),
    )(h, degp)


def _tc_mid(acc1p, g1s, degp, b1, w2):
    _, n, f2 = g1s.shape
    dh = 2 * f2
    dout = w2.shape[1]

    def body(acc_ref, g1_ref, degp_ref, b1_ref, w2_ref, g2_ref):
        dinv = _dinv_from_partials(degp_ref[...])
        acc = acc_ref[...]
        g1 = g1_ref[...]
        s = jnp.concatenate([acc[0] + g1[0], acc[1] + g1[1]], axis=1)
        h = jnp.maximum(s * dinv[:, None] + b1_ref[...][None, :], 0.0)
        g2 = jnp.dot(h, w2_ref[...], preferred_element_type=jnp.float32)
        g2_ref[...] = g2 * dinv[:, None]

    return pl.pallas_call(
        body,
        grid=(n // _BN,),
        in_specs=[
            pl.BlockSpec((NC, _BN, f2), lambda i: (0, i, 0)),
            pl.BlockSpec((NC, _BN, f2), lambda i: (0, i, 0)),
            pl.BlockSpec((NC, _BN, DEGW), lambda i: (0, i, 0)),
            pl.BlockSpec((dh,), lambda i: (0,)),
            pl.BlockSpec((dh, dout), lambda i: (0, 0)),
        ],
        out_specs=pl.BlockSpec((_BN, dout), lambda i: (i, 0)),
        out_shape=jax.ShapeDtypeStruct((n, dout), jnp.float32),
    )(acc1p, g1s, degp, b1, w2)


def _tc_final(acc2p, g2, degp, b2):
    n, dout = g2.shape

    def body(acc_ref, g2_ref, degp_ref, b2_ref, logits_ref, preds_ref, z_ref):
        dinv = _dinv_from_partials(degp_ref[...])
        acc = acc_ref[...]
        z = (acc[0] + acc[1] + g2_ref[...]) * dinv[:, None] + b2_ref[...][None, :]
        m = jnp.max(z, axis=1, keepdims=True)
        ez = jnp.exp(z - m)
        logits = ez / jnp.sum(ez, axis=1, keepdims=True)
        ids = lax.broadcasted_iota(jnp.int32, z.shape, 1)
        cand = jnp.where(z == m, ids, dout)
        z_ref[...] = z
        logits_ref[...] = logits
        preds_ref[...] = jnp.min(cand, axis=1, keepdims=True)

    return pl.pallas_call(
        body,
        grid=(n // _BN,),
        in_specs=[
            pl.BlockSpec((NC, _BN, dout), lambda i: (0, i, 0)),
            pl.BlockSpec((_BN, dout), lambda i: (i, 0)),
            pl.BlockSpec((NC, _BN, DEGW), lambda i: (0, i, 0)),
            pl.BlockSpec((dout,), lambda i: (0,)),
        ],
        out_specs=[
            pl.BlockSpec((_BN, dout), lambda i: (i, 0)),
            pl.BlockSpec((_BN, 1), lambda i: (i, 0)),
            pl.BlockSpec((_BN, dout), lambda i: (i, 0)),
        ],
        out_shape=[
            jax.ShapeDtypeStruct((n, dout), jnp.float32),
            jax.ShapeDtypeStruct((n, 1), jnp.int32),
            jax.ShapeDtypeStruct((n, dout), jnp.float32),
        ],
    )(acc2p, g2, degp, b2)


def kernel(x, edge_index, W1, b1, W2, b2):
    n = x.shape[0]
    e = edge_index.shape[1]
    nh = e // NW // CHUNK
    ei = edge_index.astype(jnp.int32)
    ei4 = ei.reshape(2, NC, NS, nh, CHUNK)
    srcp4 = (ei[0] + n).reshape(NC, NS, nh, CHUNK)

    degp = _sc_degree(ei4, n)
    h1 = _tc_matmul(x, W1)
    g1s = _tc_scale(h1, degp)
    acc1 = _sc_aggregate(g1s.reshape(NC * n, -1), ei4, srcp4, n, True)
    g2 = _tc_mid(acc1, g1s, degp, b1, W2)
    acc2 = _sc_aggregate(g2, ei4, srcp4, n, False)
    logits, preds, z = _tc_final(acc2, g2, degp, b2)
    return (logits, preds.reshape(n), z)
